# baseline (speedup 1.0000x reference)
import functools

import jax
import jax.numpy as jnp
from jax import lax
from jax.experimental import pallas as pl
from jax.experimental.pallas import tpu as pltpu
from jax.experimental.pallas import tpu_sc as plsc

N_NODES_C = 10000
NSEG = 10240
W = 48
N_INC_C = 320000
NP_INC = 327680
BLK = 128
BPG = 4
B0 = 48
B1 = 112
NIDX = 2560 + BLK
ROWS_PT = NSEG // 16
CHUNK = ROWS_PT // 2

_MESH = plsc.VectorSubcoreMesh(
    core_axis_name="c", subcore_axis_name="s", num_cores=2, num_subcores=16)

@functools.partial(
    pl.kernel,
    out_type=jax.ShapeDtypeStruct((2, NSEG, W), jnp.float32),
    mesh=_MESH,
    compiler_params=pltpu.CompilerParams(use_tc_tiling_on_sc=False),
    scratch_types=[
        pltpu.VMEM((112, BLK), jnp.int32),
        pltpu.VMEM((112, BLK), jnp.int32),
        pltpu.VMEM((BPG * BLK, W), jnp.float32),
        pltpu.VMEM((BPG * BLK, W), jnp.float32),
        pltpu.VMEM_SHARED((NSEG, W), jnp.float32),
        pltpu.SemaphoreType.DMA,
        pltpu.SemaphoreType.DMA,
        pltpu.SemaphoreType.DMA,
        pltpu.SemaphoreType.DMA,
    ],
)
def _prop(table, zrows, srcb, dstb, out,
          sidx, didx, bufa, bufb, acc, gsa, gsb, ssa, ssb):
    cid = lax.axis_index("c")
    sid = lax.axis_index("s")
    blk0 = jnp.where(cid == 0, sid * B0, 16 * B0 + sid * B1)
    pairs = jnp.where(cid == 0, B0 // (2 * BPG), B1 // (2 * BPG))

    pltpu.sync_copy(srcb.at[pl.ds(blk0, 112)], sidx)
    pltpu.sync_copy(dstb.at[pl.ds(blk0, 112)], didx)
    pltpu.sync_copy(zrows.at[pl.ds(sid * ROWS_PT, ROWS_PT)],
                    acc.at[pl.ds(sid * ROWS_PT, ROWS_PT)])
    plsc.subcore_barrier()

    def _fire_g(g, buf, sem):
        for j in range(BPG):
            pltpu.async_copy(table.at[sidx.at[g * BPG + j]],
                             buf.at[pl.ds(j * BLK, BLK)], sem)

    def _wait_g(g, buf, sem):
        for j in range(BPG):
            pltpu.make_async_copy(table.at[sidx.at[g * BPG + j]],
                                  buf.at[pl.ds(j * BLK, BLK)], sem).wait()

    def _fire_s(g, buf, sem):
        for j in range(BPG):
            pltpu.async_copy(buf.at[pl.ds(j * BLK, BLK)],
                             acc.at[didx.at[g * BPG + j]], sem, add=True)

    def _wait_s(g, buf, sem):
        for j in range(BPG):
            pltpu.make_async_copy(buf.at[pl.ds(j * BLK, BLK)],
                                  acc.at[didx.at[g * BPG + j]], sem).wait()

    _fire_g(0, bufa, gsa)

    def _pair(s, _):
        ga = 2 * s
        gb = 2 * s + 1
        _wait_g(ga, bufa, gsa)
        _fire_s(ga, bufa, ssa)

        @pl.when(s >= 1)
        def _():
            _wait_s(gb - 2, bufb, ssb)
        _fire_g(gb, bufb, gsb)
        _wait_g(gb, bufb, gsb)
        _fire_s(gb, bufb, ssb)
        _wait_s(ga, bufa, ssa)

        @pl.when(s <= pairs - 2)
        def _():
            _fire_g(ga + 2, bufa, gsa)
        return 0
    lax.fori_loop(0, pairs, _pair, 0)
    _wait_s(2 * pairs - 1, bufb, ssb)

    plsc.subcore_barrier()
    r0 = sid * ROWS_PT
    pltpu.sync_copy(acc.at[pl.ds(r0, ROWS_PT)], out.at[cid, pl.ds(r0, ROWS_PT)])


def _mm1_body(x_ref, w_ref, o_ref):
    o = jnp.dot(x_ref[...], w_ref[...], preferred_element_type=jnp.float32)
    colid = lax.broadcasted_iota(jnp.int32, o.shape, 1)
    o_ref[...] = o + jnp.where(colid == 32, 1.0, 0.0)


def _scale1_body(p_ref, ef_ref, binv_ref):
    s = p_ref[0] + p_ref[1]
    cnt = s[:, 32:33]
    binv = jnp.where(cnt > 0, 1.0 / cnt, 0.0)
    ef_ref[...] = binv * s
    binv_ref[...] = jnp.broadcast_to(binv, (NSEG, 8))


def _hidden_body(p_ref, b1_ref, w2_ref, o_ref, dinv_ref):
    s = p_ref[0] + p_ref[1]
    d = s[:, 32:33]
    dinv = jnp.where(d > 0, 1.0 / d, 0.0)
    h = jnp.maximum(dinv * s[:, :32] + b1_ref[0:1, :], 0.0)
    o_ref[...] = jnp.dot(h, w2_ref[...], preferred_element_type=jnp.float32)
    dinv_ref[...] = jnp.broadcast_to(dinv, (NSEG, 8))


def _scale2_body(p_ref, binv_ref, ef_ref):
    ef_ref[...] = binv_ref[:, 0:1] * (p_ref[0] + p_ref[1])


def _out_body(p_ref, dinv_ref, b2_ref, o_ref):
    z = dinv_ref[:, 0:1] * (p_ref[0] + p_ref[1])[:, :40] + b2_ref[0:1, :]
    m = jnp.max(z, axis=1, keepdims=True)
    e = jnp.exp(z - m)
    o_ref[...] = (z - m) - jnp.log(jnp.sum(e, axis=1, keepdims=True))


def _full(shape):
    return pl.BlockSpec(shape, lambda: (0,) * len(shape))


def kernel(x, hyperedge_index, W1, b1, W2, b2):
    f32 = jnp.float32
    row = hyperedge_index[0].astype(jnp.int32)
    col = hyperedge_index[1].astype(jnp.int32)
    pad = jnp.full((NIDX * BLK - N_INC_C,), N_NODES_C, jnp.int32)
    row_p = jnp.concatenate([row, pad]).reshape(NIDX, BLK)
    col_p = jnp.concatenate([col, pad]).reshape(NIDX, BLK)

    xpad = jnp.zeros((NSEG, 128), f32).at[:N_NODES_C].set(x)
    zrows = jnp.zeros((NSEG, W), f32)
    w1p = jnp.zeros((128, W), f32).at[:, :32].set(W1)
    w2p = jnp.zeros((32, W), f32).at[:, :40].set(W2)
    b1b = jnp.broadcast_to(b1.astype(f32), (8, 32))
    b2b = jnp.broadcast_to(b2.astype(f32), (8, 40))

    t1 = pl.pallas_call(
        _mm1_body,
        grid=(10,),
        in_specs=[pl.BlockSpec((1024, 128), lambda i: (i, 0)),
                  pl.BlockSpec((128, W), lambda i: (0, 0))],
        out_specs=pl.BlockSpec((1024, W), lambda i: (i, 0)),
        out_shape=jax.ShapeDtypeStruct((NSEG, W), f32),
    )(xpad, w1p)

    pA = _prop(t1, zrows, row_p, col_p)

    ef1, binv = pl.pallas_call(
        _scale1_body,
        in_specs=[_full((2, NSEG, W))],
        out_specs=[_full((NSEG, W)), _full((NSEG, 8))],
        out_shape=[jax.ShapeDtypeStruct((NSEG, W), f32),
                   jax.ShapeDtypeStruct((NSEG, 8), f32)],
    )(pA)

    pB = _prop(ef1, zrows, col_p, row_p)

    t2, dinv = pl.pallas_call(
        _hidden_body,
        in_specs=[_full((2, NSEG, W)), _full((8, 32)), _full((32, W))],
        out_specs=[_full((NSEG, W)), _full((NSEG, 8))],
        out_shape=[jax.ShapeDtypeStruct((NSEG, W), f32),
                   jax.ShapeDtypeStruct((NSEG, 8), f32)],
    )(pB, b1b, w2p)

    pA2 = _prop(t2, zrows, row_p, col_p)

    ef2 = pl.pallas_call(
        _scale2_body,
        in_specs=[_full((2, NSEG, W)), _full((NSEG, 8))],
        out_specs=_full((NSEG, W)),
        out_shape=jax.ShapeDtypeStruct((NSEG, W), f32),
    )(pA2, binv)

    pB2 = _prop(ef2, zrows, col_p, row_p)

    out = pl.pallas_call(
        _out_body,
        in_specs=[_full((2, NSEG, W)), _full((NSEG, 8)), _full((8, 40))],
        out_specs=_full((NSEG, 40)),
        out_shape=jax.ShapeDtypeStruct((NSEG, 40), f32),
    )(pB2, dinv, b2b)

    return out[:N_NODES_C]


# speedup vs baseline: 2.3581x; 2.3581x over previous
import functools

import jax
import jax.numpy as jnp
from jax import lax
from jax.experimental import pallas as pl
from jax.experimental.pallas import tpu as pltpu
from jax.experimental.pallas import tpu_sc as plsc

N_NODES_C = 10000
NSEG = 10240
W = 48
N_INC_C = 320000
NP_INC = 327680
BLK = 128
BPG = 4
NBLK = 80
PAIRS = NBLK // (2 * BPG)
ROWS_PT = NSEG // 16
CHUNK = ROWS_PT // 2

_MESH = plsc.VectorSubcoreMesh(
    core_axis_name="c", subcore_axis_name="s", num_cores=2, num_subcores=16)

@functools.partial(
    pl.kernel,
    out_type=jax.ShapeDtypeStruct((2, NSEG, W), jnp.float32),
    mesh=_MESH,
    compiler_params=pltpu.CompilerParams(use_tc_tiling_on_sc=False),
    scratch_types=[
        pltpu.VMEM((NBLK, BLK), jnp.int32),
        pltpu.VMEM((NBLK, BLK), jnp.int32),
        pltpu.VMEM((BPG * BLK, W), jnp.float32),
        pltpu.VMEM((BPG * BLK, W), jnp.float32),
        pltpu.VMEM_SHARED((NSEG, W), jnp.float32),
        pltpu.VMEM_SHARED((NSEG, W), jnp.float32),
        pltpu.SemaphoreType.DMA,
        pltpu.SemaphoreType.DMA,
        pltpu.SemaphoreType.DMA,
        pltpu.SemaphoreType.DMA,
    ],
)
def _prop(table, zrows, srcb, dstb, out,
          sidx, didx, bufa, bufb, acc, tbl, gsa, gsb, ssa, ssb):
    cid = lax.axis_index("c")
    sid = lax.axis_index("s")
    wid = cid * 16 + sid
    blk0 = wid * NBLK

    pltpu.sync_copy(srcb.at[pl.ds(blk0, NBLK)], sidx)
    pltpu.sync_copy(dstb.at[pl.ds(blk0, NBLK)], didx)
    pltpu.sync_copy(zrows.at[pl.ds(sid * ROWS_PT, ROWS_PT)],
                    acc.at[pl.ds(sid * ROWS_PT, ROWS_PT)])
    pltpu.sync_copy(table.at[pl.ds(sid * ROWS_PT, ROWS_PT)],
                    tbl.at[pl.ds(sid * ROWS_PT, ROWS_PT)])
    plsc.subcore_barrier()

    def _fire_g(g, buf, sem):
        for j in range(BPG):
            pltpu.async_copy(tbl.at[sidx.at[g * BPG + j]],
                             buf.at[pl.ds(j * BLK, BLK)], sem)

    def _wait_g(g, buf, sem):
        for j in range(BPG):
            pltpu.make_async_copy(tbl.at[sidx.at[g * BPG + j]],
                                  buf.at[pl.ds(j * BLK, BLK)], sem).wait()

    def _fire_s(g, buf, sem):
        for j in range(BPG):
            pltpu.async_copy(buf.at[pl.ds(j * BLK, BLK)],
                             acc.at[didx.at[g * BPG + j]], sem, add=True)

    def _wait_s(g, buf, sem):
        for j in range(BPG):
            pltpu.make_async_copy(buf.at[pl.ds(j * BLK, BLK)],
                                  acc.at[didx.at[g * BPG + j]], sem).wait()

    _fire_g(0, bufa, gsa)

    def _pair(s, _):
        ga = 2 * s
        gb = 2 * s + 1
        _wait_g(ga, bufa, gsa)
        _fire_s(ga, bufa, ssa)

        @pl.when(s >= 1)
        def _():
            _wait_s(gb - 2, bufb, ssb)
        _fire_g(gb, bufb, gsb)
        _wait_g(gb, bufb, gsb)
        _fire_s(gb, bufb, ssb)
        _wait_s(ga, bufa, ssa)

        @pl.when(s <= PAIRS - 2)
        def _():
            _fire_g(ga + 2, bufa, gsa)
        return 0
    lax.fori_loop(0, PAIRS, _pair, 0)
    _wait_s(2 * PAIRS - 1, bufb, ssb)

    plsc.subcore_barrier()
    r0 = sid * ROWS_PT
    pltpu.sync_copy(acc.at[pl.ds(r0, ROWS_PT)], out.at[cid, pl.ds(r0, ROWS_PT)])


def _mm1_body(x_ref, w_ref, o_ref):
    o = jnp.dot(x_ref[...], w_ref[...], preferred_element_type=jnp.float32)
    colid = lax.broadcasted_iota(jnp.int32, o.shape, 1)
    o_ref[...] = o + jnp.where(colid == 32, 1.0, 0.0)


def _scale1_body(p_ref, ef_ref, binv_ref):
    s = p_ref[0] + p_ref[1]
    cnt = s[:, 32:33]
    binv = jnp.where(cnt > 0, 1.0 / cnt, 0.0)
    ef_ref[...] = binv * s
    binv_ref[...] = jnp.broadcast_to(binv, (NSEG, 8))


def _hidden_body(p_ref, b1_ref, w2_ref, o_ref, dinv_ref):
    s = p_ref[0] + p_ref[1]
    d = s[:, 32:33]
    dinv = jnp.where(d > 0, 1.0 / d, 0.0)
    h = jnp.maximum(dinv * s[:, :32] + b1_ref[0:1, :], 0.0)
    o_ref[...] = jnp.dot(h, w2_ref[...], preferred_element_type=jnp.float32)
    dinv_ref[...] = jnp.broadcast_to(dinv, (NSEG, 8))


def _scale2_body(p_ref, binv_ref, ef_ref):
    ef_ref[...] = binv_ref[:, 0:1] * (p_ref[0] + p_ref[1])


def _out_body(p_ref, dinv_ref, b2_ref, o_ref):
    z = dinv_ref[:, 0:1] * (p_ref[0] + p_ref[1])[:, :40] + b2_ref[0:1, :]
    m = jnp.max(z, axis=1, keepdims=True)
    e = jnp.exp(z - m)
    o_ref[...] = (z - m) - jnp.log(jnp.sum(e, axis=1, keepdims=True))


def _full(shape):
    return pl.BlockSpec(shape, lambda: (0,) * len(shape))


def kernel(x, hyperedge_index, W1, b1, W2, b2):
    f32 = jnp.float32
    row = hyperedge_index[0].astype(jnp.int32)
    col = hyperedge_index[1].astype(jnp.int32)
    pad = jnp.full((NP_INC - N_INC_C,), N_NODES_C, jnp.int32)
    row_p = jnp.concatenate([row, pad]).reshape(NP_INC // BLK, BLK)
    col_p = jnp.concatenate([col, pad]).reshape(NP_INC // BLK, BLK)

    xpad = jnp.zeros((NSEG, 128), f32).at[:N_NODES_C].set(x)
    zrows = jnp.zeros((NSEG, W), f32)
    w1p = jnp.zeros((128, W), f32).at[:, :32].set(W1)
    w2p = jnp.zeros((32, W), f32).at[:, :40].set(W2)
    b1b = jnp.broadcast_to(b1.astype(f32), (8, 32))
    b2b = jnp.broadcast_to(b2.astype(f32), (8, 40))

    t1 = pl.pallas_call(
        _mm1_body,
        grid=(10,),
        in_specs=[pl.BlockSpec((1024, 128), lambda i: (i, 0)),
                  pl.BlockSpec((128, W), lambda i: (0, 0))],
        out_specs=pl.BlockSpec((1024, W), lambda i: (i, 0)),
        out_shape=jax.ShapeDtypeStruct((NSEG, W), f32),
    )(xpad, w1p)

    pA = _prop(t1, zrows, row_p, col_p)

    ef1, binv = pl.pallas_call(
        _scale1_body,
        in_specs=[_full((2, NSEG, W))],
        out_specs=[_full((NSEG, W)), _full((NSEG, 8))],
        out_shape=[jax.ShapeDtypeStruct((NSEG, W), f32),
                   jax.ShapeDtypeStruct((NSEG, 8), f32)],
    )(pA)

    pB = _prop(ef1, zrows, col_p, row_p)

    t2, dinv = pl.pallas_call(
        _hidden_body,
        in_specs=[_full((2, NSEG, W)), _full((8, 32)), _full((32, W))],
        out_specs=[_full((NSEG, W)), _full((NSEG, 8))],
        out_shape=[jax.ShapeDtypeStruct((NSEG, W), f32),
                   jax.ShapeDtypeStruct((NSEG, 8), f32)],
    )(pB, b1b, w2p)

    pA2 = _prop(t2, zrows, row_p, col_p)

    ef2 = pl.pallas_call(
        _scale2_body,
        in_specs=[_full((2, NSEG, W)), _full((NSEG, 8))],
        out_specs=_full((NSEG, W)),
        out_shape=jax.ShapeDtypeStruct((NSEG, W), f32),
    )(pA2, binv)

    pB2 = _prop(ef2, zrows, col_p, row_p)

    out = pl.pallas_call(
        _out_body,
        in_specs=[_full((2, NSEG, W)), _full((NSEG, 8)), _full((8, 40))],
        out_specs=_full((NSEG, 40)),
        out_shape=jax.ShapeDtypeStruct((NSEG, 40), f32),
    )(pB2, dinv, b2b)

    return out[:N_NODES_C]


# speedup vs baseline: 2.4035x; 1.0192x over previous
import functools

import jax
import jax.numpy as jnp
from jax import lax
from jax.experimental import pallas as pl
from jax.experimental.pallas import tpu as pltpu
from jax.experimental.pallas import tpu_sc as plsc

N_NODES_C = 10000
NSEG = 10240
W = 48
N_INC_C = 320000
NP_INC = 327680
BLK = 128
BPG = 4
NBLK = 80
PAIRS = NBLK // (2 * BPG)
ROWS_PT = NSEG // 16
CHUNK = ROWS_PT // 2

_MESH = plsc.VectorSubcoreMesh(
    core_axis_name="c", subcore_axis_name="s", num_cores=2, num_subcores=16)

@functools.partial(
    pl.kernel,
    out_type=jax.ShapeDtypeStruct((2, NSEG, W), jnp.float32),
    mesh=_MESH,
    compiler_params=pltpu.CompilerParams(use_tc_tiling_on_sc=False),
    scratch_types=[
        pltpu.VMEM((NBLK, BLK), jnp.int32),
        pltpu.VMEM((NBLK, BLK), jnp.int32),
        pltpu.VMEM((BPG * BLK, W), jnp.float32),
        pltpu.VMEM((BPG * BLK, W), jnp.float32),
        pltpu.VMEM_SHARED((NSEG, W), jnp.float32),
        pltpu.VMEM_SHARED((NSEG, W), jnp.float32),
        pltpu.SemaphoreType.DMA,
        pltpu.SemaphoreType.DMA,
        pltpu.SemaphoreType.DMA,
        pltpu.SemaphoreType.DMA,
    ],
)
def _prop(table, zrows, srcb, dstb, out,
          sidx, didx, bufa, bufb, acc, tbl, gsa, gsb, ssa, ssb):
    cid = lax.axis_index("c")
    sid = lax.axis_index("s")
    wid = cid * 16 + sid
    blk0 = wid * NBLK

    r0p = sid * ROWS_PT
    d1 = pltpu.async_copy(srcb.at[pl.ds(blk0, NBLK)], sidx, gsa)
    d2 = pltpu.async_copy(dstb.at[pl.ds(blk0, NBLK)], didx, gsa)
    d3 = pltpu.async_copy(zrows.at[pl.ds(r0p, ROWS_PT)],
                          acc.at[pl.ds(r0p, ROWS_PT)], gsa)
    d4 = pltpu.async_copy(table.at[pl.ds(r0p, ROWS_PT)],
                          tbl.at[pl.ds(r0p, ROWS_PT)], gsa)
    d1.wait()
    d2.wait()
    d3.wait()
    d4.wait()
    plsc.subcore_barrier()

    def _fire_g(g, buf, sem):
        for j in range(BPG):
            pltpu.async_copy(tbl.at[sidx.at[g * BPG + j]],
                             buf.at[pl.ds(j * BLK, BLK)], sem)

    def _wait_g(g, buf, sem):
        for j in range(BPG):
            pltpu.make_async_copy(tbl.at[sidx.at[g * BPG + j]],
                                  buf.at[pl.ds(j * BLK, BLK)], sem).wait()

    def _fire_s(g, buf, sem):
        for j in range(BPG):
            pltpu.async_copy(buf.at[pl.ds(j * BLK, BLK)],
                             acc.at[didx.at[g * BPG + j]], sem, add=True)

    def _wait_s(g, buf, sem):
        for j in range(BPG):
            pltpu.make_async_copy(buf.at[pl.ds(j * BLK, BLK)],
                                  acc.at[didx.at[g * BPG + j]], sem).wait()

    _fire_g(0, bufa, gsa)

    def _pair(s, _):
        ga = 2 * s
        gb = 2 * s + 1
        _wait_g(ga, bufa, gsa)
        _fire_s(ga, bufa, ssa)

        @pl.when(s >= 1)
        def _():
            _wait_s(gb - 2, bufb, ssb)
        _fire_g(gb, bufb, gsb)
        _wait_g(gb, bufb, gsb)
        _fire_s(gb, bufb, ssb)
        _wait_s(ga, bufa, ssa)

        @pl.when(s <= PAIRS - 2)
        def _():
            _fire_g(ga + 2, bufa, gsa)
        return 0
    lax.fori_loop(0, PAIRS, _pair, 0)
    _wait_s(2 * PAIRS - 1, bufb, ssb)

    plsc.subcore_barrier()
    r0 = sid * ROWS_PT
    pltpu.sync_copy(acc.at[pl.ds(r0, ROWS_PT)], out.at[cid, pl.ds(r0, ROWS_PT)])


def _mm1_body(x_ref, w_ref, o_ref):
    o = jnp.dot(x_ref[...], w_ref[...], preferred_element_type=jnp.float32)
    colid = lax.broadcasted_iota(jnp.int32, o.shape, 1)
    o_ref[...] = o + jnp.where(colid == 32, 1.0, 0.0)


def _scale1_body(p_ref, ef_ref, binv_ref):
    s = p_ref[0] + p_ref[1]
    cnt = s[:, 32:33]
    binv = jnp.where(cnt > 0, 1.0 / cnt, 0.0)
    ef_ref[...] = binv * s
    binv_ref[...] = jnp.broadcast_to(binv, (NSEG, 8))


def _hidden_body(p_ref, b1_ref, w2_ref, o_ref, dinv_ref):
    s = p_ref[0] + p_ref[1]
    d = s[:, 32:33]
    dinv = jnp.where(d > 0, 1.0 / d, 0.0)
    h = jnp.maximum(dinv * s[:, :32] + b1_ref[0:1, :], 0.0)
    o_ref[...] = jnp.dot(h, w2_ref[...], preferred_element_type=jnp.float32)
    dinv_ref[...] = jnp.broadcast_to(dinv, (NSEG, 8))


def _scale2_body(p_ref, binv_ref, ef_ref):
    ef_ref[...] = binv_ref[:, 0:1] * (p_ref[0] + p_ref[1])


def _out_body(p_ref, dinv_ref, b2_ref, o_ref):
    z = dinv_ref[:, 0:1] * (p_ref[0] + p_ref[1])[:, :40] + b2_ref[0:1, :]
    m = jnp.max(z, axis=1, keepdims=True)
    e = jnp.exp(z - m)
    o_ref[...] = (z - m) - jnp.log(jnp.sum(e, axis=1, keepdims=True))


def _full(shape):
    return pl.BlockSpec(shape, lambda: (0,) * len(shape))


def kernel(x, hyperedge_index, W1, b1, W2, b2):
    f32 = jnp.float32
    row = hyperedge_index[0].astype(jnp.int32)
    col = hyperedge_index[1].astype(jnp.int32)
    pad = jnp.full((NP_INC - N_INC_C,), N_NODES_C, jnp.int32)
    row_p = jnp.concatenate([row, pad]).reshape(NP_INC // BLK, BLK)
    col_p = jnp.concatenate([col, pad]).reshape(NP_INC // BLK, BLK)

    xpad = jnp.zeros((NSEG, 128), f32).at[:N_NODES_C].set(x)
    zrows = jnp.zeros((NSEG, W), f32)
    w1p = jnp.zeros((128, W), f32).at[:, :32].set(W1)
    w2p = jnp.zeros((32, W), f32).at[:, :40].set(W2)
    b1b = jnp.broadcast_to(b1.astype(f32), (8, 32))
    b2b = jnp.broadcast_to(b2.astype(f32), (8, 40))

    t1 = pl.pallas_call(
        _mm1_body,
        grid=(10,),
        in_specs=[pl.BlockSpec((1024, 128), lambda i: (i, 0)),
                  pl.BlockSpec((128, W), lambda i: (0, 0))],
        out_specs=pl.BlockSpec((1024, W), lambda i: (i, 0)),
        out_shape=jax.ShapeDtypeStruct((NSEG, W), f32),
    )(xpad, w1p)

    pA = _prop(t1, zrows, row_p, col_p)

    ef1, binv = pl.pallas_call(
        _scale1_body,
        in_specs=[_full((2, NSEG, W))],
        out_specs=[_full((NSEG, W)), _full((NSEG, 8))],
        out_shape=[jax.ShapeDtypeStruct((NSEG, W), f32),
                   jax.ShapeDtypeStruct((NSEG, 8), f32)],
    )(pA)

    pB = _prop(ef1, zrows, col_p, row_p)

    t2, dinv = pl.pallas_call(
        _hidden_body,
        in_specs=[_full((2, NSEG, W)), _full((8, 32)), _full((32, W))],
        out_specs=[_full((NSEG, W)), _full((NSEG, 8))],
        out_shape=[jax.ShapeDtypeStruct((NSEG, W), f32),
                   jax.ShapeDtypeStruct((NSEG, 8), f32)],
    )(pB, b1b, w2p)

    pA2 = _prop(t2, zrows, row_p, col_p)

    ef2 = pl.pallas_call(
        _scale2_body,
        in_specs=[_full((2, NSEG, W)), _full((NSEG, 8))],
        out_specs=_full((NSEG, W)),
        out_shape=jax.ShapeDtypeStruct((NSEG, W), f32),
    )(pA2, binv)

    pB2 = _prop(ef2, zrows, col_p, row_p)

    out = pl.pallas_call(
        _out_body,
        in_specs=[_full((2, NSEG, W)), _full((NSEG, 8)), _full((8, 40))],
        out_specs=_full((NSEG, 40)),
        out_shape=jax.ShapeDtypeStruct((NSEG, 40), f32),
    )(pB2, dinv, b2b)

    return out[:N_NODES_C]
